# baseline (speedup 1.0000x reference)
import functools

import jax
import jax.numpy as jnp
from jax import lax
from jax.experimental import pallas as pl
from jax.experimental.pallas import tpu as pltpu
from jax.experimental.pallas import tpu_sc as plsc

_MAX_LEN = 512
_LANES = 16


def _positional_encodings(max_len, embed_dim):
    pos = jnp.arange(0, max_len, dtype=jnp.float32).reshape(-1, 1)
    skip = jnp.arange(0, embed_dim, 2, dtype=jnp.float32)
    denom = 10000.0 ** (skip / embed_dim)
    enc = jnp.zeros((max_len, embed_dim), dtype=jnp.float32)
    enc = enc.at[:, 0::2].set(jnp.sin(pos / denom))
    enc = enc.at[:, 1::2].set(jnp.cos(pos / denom))
    return enc


def kernel(input_ids, src_table):
    B, L = input_ids.shape
    V, D = src_table.shape
    ids = input_ids.astype(jnp.int32)
    enc = _positional_encodings(_MAX_LEN, D)[:L].astype(jnp.float32)

    info = plsc.get_sparse_core_info()
    NC, NS = info.num_cores, info.num_subcores
    NW = NC * NS
    assert B % NW == 0, (B, NW)
    rows_per_w = B // NW
    assert rows_per_w % 2 == 0
    assert D % _LANES == 0
    splits = [(o, min(128, L - o)) for o in range(0, L, 128)]

    mesh = plsc.VectorSubcoreMesh(core_axis_name="c", subcore_axis_name="s")

    @functools.partial(
        pl.kernel,
        mesh=mesh,
        compiler_params=pltpu.CompilerParams(use_tc_tiling_on_sc=False),
        out_type=jax.ShapeDtypeStruct((B, L, D), jnp.float32),
        scratch_types=[
            pltpu.VMEM((2, L), jnp.int32),
            pltpu.VMEM((2, L, D), jnp.float32),
            pltpu.VMEM((L, D), jnp.float32),
            pltpu.SemaphoreType.DMA,
            pltpu.SemaphoreType.DMA,
        ],
    )
    def run(ids_hbm, table_hbm, enc_hbm, out_hbm, idx_v, rows_v, enc_v,
            sem0, sem1):
        wid = lax.axis_index("s") * NC + lax.axis_index("c")
        row0 = wid * rows_per_w
        sems = (sem0, sem1)

        pltpu.sync_copy(enc_hbm, enc_v)

        def issue(g, b):
            r = row0 + g
            pltpu.sync_copy(ids_hbm.at[r], idx_v.at[b])
            for (o, n) in splits:
                pltpu.async_copy(
                    table_hbm.at[idx_v.at[b, pl.ds(o, n)]],
                    rows_v.at[b, pl.ds(o, n)],
                    sems[b],
                )

        def process(g, b):
            r = row0 + g
            pltpu.make_async_copy(
                table_hbm.at[pl.ds(0, L)], rows_v.at[b], sems[b]).wait()

            def add_body(i, carry):
                i0 = i * 4
                for dr in range(4):
                    for k in range(D // _LANES):
                        sl = pl.ds(k * _LANES, _LANES)
                        rows_v[b, i0 + dr, sl] = (
                            rows_v[b, i0 + dr, sl] + enc_v[i0 + dr, sl])
                return carry

            lax.fori_loop(0, L // 4, add_body, 0)
            rem = L % 4
            if rem:
                for dr in range(rem):
                    for k in range(D // _LANES):
                        sl = pl.ds(k * _LANES, _LANES)
                        rows_v[b, L - rem + dr, sl] = (
                            rows_v[b, L - rem + dr, sl]
                            + enc_v[L - rem + dr, sl])
            pltpu.sync_copy(rows_v.at[b], out_hbm.at[r])

        issue(0, 0)
        issue(1, 1)

        def outer(t, carry):
            g = t * 2
            process(g, 0)

            @pl.when(g + 2 < rows_per_w)
            def _():
                issue(g + 2, 0)

            process(g + 1, 1)

            @pl.when(g + 3 < rows_per_w)
            def _():
                issue(g + 3, 1)

            return carry

        lax.fori_loop(0, rows_per_w // 2, outer, 0)

    return run(ids, src_table, enc)


# speedup vs baseline: 1.0918x; 1.0918x over previous
import functools

import jax
import jax.numpy as jnp
from jax import lax
from jax.experimental import pallas as pl
from jax.experimental.pallas import tpu as pltpu
from jax.experimental.pallas import tpu_sc as plsc

_MAX_LEN = 512
_LANES = 16
_NBUF = 4


def _positional_encodings(max_len, embed_dim):
    pos = jnp.arange(0, max_len, dtype=jnp.float32).reshape(-1, 1)
    skip = jnp.arange(0, embed_dim, 2, dtype=jnp.float32)
    denom = 10000.0 ** (skip / embed_dim)
    enc = jnp.zeros((max_len, embed_dim), dtype=jnp.float32)
    enc = enc.at[:, 0::2].set(jnp.sin(pos / denom))
    enc = enc.at[:, 1::2].set(jnp.cos(pos / denom))
    return enc


def kernel(input_ids, src_table):
    B, L = input_ids.shape
    V, D = src_table.shape
    ids = input_ids.astype(jnp.int32)
    enc = _positional_encodings(_MAX_LEN, D)[:L].astype(jnp.float32)

    info = plsc.get_sparse_core_info()
    NC, NS = info.num_cores, info.num_subcores
    NW = NC * NS
    assert B % NW == 0, (B, NW)
    rows_per_w = B // NW
    assert rows_per_w % _NBUF == 0
    assert D % _LANES == 0
    splits = [(o, min(128, L - o)) for o in range(0, L, 128)]

    mesh = plsc.VectorSubcoreMesh(core_axis_name="c", subcore_axis_name="s")

    @functools.partial(
        pl.kernel,
        mesh=mesh,
        compiler_params=pltpu.CompilerParams(use_tc_tiling_on_sc=False),
        out_type=jax.ShapeDtypeStruct((B, L, D), jnp.float32),
        scratch_types=[
            pltpu.VMEM((rows_per_w, L), jnp.int32),
            pltpu.VMEM((_NBUF, L, D), jnp.float32),
            pltpu.VMEM((L, D), jnp.float32),
            [pltpu.SemaphoreType.DMA] * _NBUF,
            [pltpu.SemaphoreType.DMA] * _NBUF,
        ],
    )
    def run(ids_hbm, table_hbm, enc_hbm, out_hbm, idx_all, rows_v, enc_v,
            gsems, osems):
        wid = lax.axis_index("s") * NC + lax.axis_index("c")
        row0 = wid * rows_per_w

        pltpu.sync_copy(enc_hbm, enc_v)
        pltpu.sync_copy(ids_hbm.at[pl.ds(row0, rows_per_w)], idx_all)

        def start_gather(g, b):
            for (o, n) in splits:
                pltpu.async_copy(
                    table_hbm.at[idx_all.at[g, pl.ds(o, n)]],
                    rows_v.at[b, pl.ds(o, n)],
                    gsems[b],
                )

        def wait_gather(b):
            pltpu.make_async_copy(
                table_hbm.at[pl.ds(0, L)], rows_v.at[b], gsems[b]).wait()

        def wait_out(b):
            pltpu.make_async_copy(
                rows_v.at[b], out_hbm.at[row0], osems[b]).wait()

        def add_enc(b):
            def add_body(i, carry):
                i0 = i * 4
                for dr in range(4):
                    for k in range(D // _LANES):
                        sl = pl.ds(k * _LANES, _LANES)
                        rows_v[b, i0 + dr, sl] = (
                            rows_v[b, i0 + dr, sl] + enc_v[i0 + dr, sl])
                return carry

            lax.fori_loop(0, L // 4, add_body, 0)
            rem = L % 4
            for dr in range(rem):
                for k in range(D // _LANES):
                    sl = pl.ds(k * _LANES, _LANES)
                    rows_v[b, L - rem + dr, sl] = (
                        rows_v[b, L - rem + dr, sl] + enc_v[L - rem + dr, sl])

        for b in range(_NBUF):
            start_gather(b, b)

        def outer(t, carry):
            for b in range(_NBUF):
                g = t * _NBUF + b
                bp = (b - 1) % _NBUF
                g_next = g + _NBUF - 1
                wait_gather(b)
                add_enc(b)

                @pl.when(jnp.logical_and(g_next >= _NBUF,
                                         g_next < rows_per_w))
                def _():
                    wait_out(bp)
                    start_gather(g_next, bp)

                pltpu.async_copy(rows_v.at[b], out_hbm.at[row0 + g], osems[b])
            return carry

        lax.fori_loop(0, rows_per_w // _NBUF, outer, 0)

        for b in range(_NBUF):
            wait_out(b)

    return run(ids, src_table, enc)
